# baseline (speedup 1.0000x reference)
import functools

import jax
import jax.numpy as jnp
from jax import lax
from jax.experimental import pallas as pl
from jax.experimental.pallas import tpu as pltpu
from jax.experimental.pallas import tpu_sc as plsc

B = 4
M = 2048
NC, NS, L = 2, 16, 16
NW = NC * NS
W_PER_B = NW // B

SC_X = 512
SC_ROWS_PER_W = B * SC_X // NW
CHUNK = 4
PASS_ROWS = CHUNK * L
NPASS = SC_ROWS_PER_W // PASS_ROWS

TC_RB = 512
TC_CB = 512
TC_NRB = (M - SC_X) // TC_RB


def _sc_nn_cost(p1x, p1y, p2x, p2y):
    mesh = plsc.VectorSubcoreMesh(
        core_axis_name="c", subcore_axis_name="s",
        num_cores=NC, num_subcores=NS)

    @functools.partial(
        pl.kernel,
        out_type=jax.ShapeDtypeStruct((NW * L,), jnp.float32),
        mesh=mesh,
        scratch_types=[
            pltpu.VMEM((SC_ROWS_PER_W,), jnp.float32),
            pltpu.VMEM((SC_ROWS_PER_W,), jnp.float32),
            pltpu.VMEM((M,), jnp.float32),
            pltpu.VMEM((M,), jnp.float32),
            pltpu.VMEM((M,), jnp.float32),
            pltpu.VMEM((L,), jnp.float32),
        ],
    )
    def k(p1x_h, p1y_h, p2x_h, p2y_h, out_h,
          p1x_v, p1y_v, p2x_v, p2y_v, n2_v, out_v):
        c = lax.axis_index("c")
        s = lax.axis_index("s")
        w = c * NS + s
        b = w // W_PER_B
        base = b * M + (w % W_PER_B) * SC_ROWS_PER_W
        pltpu.sync_copy(p1x_h.at[pl.ds(base, SC_ROWS_PER_W)], p1x_v)
        pltpu.sync_copy(p1y_h.at[pl.ds(base, SC_ROWS_PER_W)], p1y_v)
        pltpu.sync_copy(p2x_h.at[pl.ds(b * M, M)], p2x_v)
        pltpu.sync_copy(p2y_h.at[pl.ds(b * M, M)], p2y_v)

        def n2_body(kk, carry):
            x2 = p2x_v[pl.ds(kk * L, L)]
            y2 = p2y_v[pl.ds(kk * L, L)]
            n2_v[pl.ds(kk * L, L)] = x2 * x2 + y2 * y2
            return carry
        lax.fori_loop(0, M // L, n2_body, 0)

        s_vec = jnp.zeros((L,), jnp.float32)
        for p in range(NPASS):
            nx, ny = [], []
            for rc in range(CHUNK):
                off = p * PASS_ROWS + rc * L
                x1 = p1x_v[pl.ds(off, L)]
                y1 = p1y_v[pl.ds(off, L)]
                s_vec = s_vec + (x1 * x1 + y1 * y1)
                nx.append(x1 * -2.0)
                ny.append(y1 * -2.0)

            def jj_body(jj, m, nx=nx, ny=ny):
                base_j = jj * L
                x2v = p2x_v[pl.ds(base_j, L)]
                y2v = p2y_v[pl.ds(base_j, L)]
                n2v = n2_v[pl.ds(base_j, L)]

                def u_body(u, mm):
                    uv = jnp.full((L,), u, jnp.int32)
                    x2b = jnp.take_along_axis(
                        x2v, uv, axis=0, mode="promise_in_bounds")
                    y2b = jnp.take_along_axis(
                        y2v, uv, axis=0, mode="promise_in_bounds")
                    n2b = jnp.take_along_axis(
                        n2v, uv, axis=0, mode="promise_in_bounds")
                    return tuple(
                        jnp.minimum(mm[rc], n2b + nx[rc] * x2b + ny[rc] * y2b)
                        for rc in range(CHUNK))

                return lax.fori_loop(0, L, u_body, m, unroll=2)

            m0 = tuple(jnp.full((L,), 3.0e38, jnp.float32)
                       for _ in range(CHUNK))
            m = lax.fori_loop(0, M // L, jj_body, m0)
            for rc in range(CHUNK):
                s_vec = s_vec + m[rc]

        out_v[...] = s_vec
        pltpu.sync_copy(out_v, out_h.at[pl.ds(w * L, L)])

    return k(p1x, p1y, p2x, p2y)


def _tc_body(p1x_r, p1y_r, p2x_r, p2y_r, o_r):
    x2 = p2x_r[0, 0, :]
    y2 = p2y_r[0, 0, :]
    n2 = (x2 * x2 + y2 * y2)[None, :]
    bmat = jnp.concatenate([x2[None, :], y2[None, :]], axis=0)
    s = jnp.float32(0.0)
    for rb in range(TC_NRB):
        r0 = SC_X + rb * TC_RB
        x1 = p1x_r[0, 0, pl.ds(r0, TC_RB)]
        y1 = p1y_r[0, 0, pl.ds(r0, TC_RB)]
        a = jnp.concatenate([(x1 * -2.0)[:, None],
                             (y1 * -2.0)[:, None]], axis=1)
        g = lax.dot_general(a, bmat, (((1,), (0,)), ((), ())),
                            precision=lax.Precision.HIGHEST,
                            preferred_element_type=jnp.float32)
        m = jnp.min(n2 + g, axis=1)
        s = s + jnp.sum(m + (x1 * x1 + y1 * y1))
    o_r[0, 0, :] = jnp.full((128,), s, jnp.float32)


def _tc_nn_cost(p1x2, p1y2, p2x2, p2y2):
    a = [v.reshape(B, 1, M) for v in (p1x2, p1y2, p2x2, p2y2)]
    spec = pl.BlockSpec((1, 1, M), lambda b: (b, 0, 0))
    out = pl.pallas_call(
        _tc_body,
        grid=(B,),
        in_specs=[spec, spec, spec, spec],
        out_specs=pl.BlockSpec((1, 1, 128), lambda b: (b, 0, 0)),
        out_shape=jax.ShapeDtypeStruct((B, 1, 128), jnp.float32),
        compiler_params=pltpu.CompilerParams(
            dimension_semantics=("parallel",)),
    )(*a)
    return out[:, 0, 0]


@jax.jit
def kernel(p1, p2):
    p1x2 = p1[:, :, 0]
    p1y2 = p1[:, :, 1]
    p2x2 = p2[:, :, 0]
    p2y2 = p2[:, :, 1]
    tc_part = _tc_nn_cost(p1x2, p1y2, p2x2, p2y2)
    sc_part = _sc_nn_cost(p1x2.reshape(-1), p1y2.reshape(-1),
                          p2x2.reshape(-1), p2y2.reshape(-1))
    return sc_part.reshape(B, W_PER_B * L).sum(axis=1) + tc_part


# speedup vs baseline: 1.2344x; 1.2344x over previous
import functools

import jax
import jax.numpy as jnp
from jax import lax
from jax.experimental import pallas as pl
from jax.experimental.pallas import tpu as pltpu
from jax.experimental.pallas import tpu_sc as plsc

B = 4
M = 2048
NC, NS, L = 2, 16, 16
NW = NC * NS
W_PER_B = NW // B

SC_X = 512
SC_ROWS_PER_W = B * SC_X // NW
CHUNK = 4
PASS_ROWS = CHUNK * L
NPASS = SC_ROWS_PER_W // PASS_ROWS

TC_RB = 512
TC_CB = 512
TC_NRB = (M - SC_X) // TC_RB


def _sc_nn_cost(p1x, p1y, p2x, p2y):
    mesh = plsc.VectorSubcoreMesh(
        core_axis_name="c", subcore_axis_name="s",
        num_cores=NC, num_subcores=NS)

    @functools.partial(
        pl.kernel,
        out_type=jax.ShapeDtypeStruct((NW * L,), jnp.float32),
        mesh=mesh,
        scratch_types=[
            pltpu.VMEM((SC_ROWS_PER_W,), jnp.float32),
            pltpu.VMEM((SC_ROWS_PER_W,), jnp.float32),
            pltpu.VMEM((M,), jnp.float32),
            pltpu.VMEM((M,), jnp.float32),
            pltpu.VMEM((M,), jnp.float32),
            pltpu.VMEM((L,), jnp.float32),
        ],
    )
    def k(p1x_h, p1y_h, p2x_h, p2y_h, out_h,
          p1x_v, p1y_v, p2x_v, p2y_v, n2_v, out_v):
        c = lax.axis_index("c")
        s = lax.axis_index("s")
        w = c * NS + s
        b = w // W_PER_B
        base = b * M + (w % W_PER_B) * SC_ROWS_PER_W
        pltpu.sync_copy(p1x_h.at[pl.ds(base, SC_ROWS_PER_W)], p1x_v)
        pltpu.sync_copy(p1y_h.at[pl.ds(base, SC_ROWS_PER_W)], p1y_v)
        pltpu.sync_copy(p2x_h.at[pl.ds(b * M, M)], p2x_v)
        pltpu.sync_copy(p2y_h.at[pl.ds(b * M, M)], p2y_v)

        def n2_body(kk, carry):
            x2 = p2x_v[pl.ds(kk * L, L)]
            y2 = p2y_v[pl.ds(kk * L, L)]
            n2_v[pl.ds(kk * L, L)] = x2 * x2 + y2 * y2
            return carry
        lax.fori_loop(0, M // L, n2_body, 0)

        s_vec = jnp.zeros((L,), jnp.float32)
        for p in range(NPASS):
            nx, ny = [], []
            for rc in range(CHUNK):
                off = p * PASS_ROWS + rc * L
                x1 = p1x_v[pl.ds(off, L)]
                y1 = p1y_v[pl.ds(off, L)]
                s_vec = s_vec + (x1 * x1 + y1 * y1)
                nx.append(x1 * -2.0)
                ny.append(y1 * -2.0)

            def jj_body(jj, m, nx=nx, ny=ny):
                base_j = jj * L
                x2v = p2x_v[pl.ds(base_j, L)]
                y2v = p2y_v[pl.ds(base_j, L)]
                n2v = n2_v[pl.ds(base_j, L)]

                def u_body(u, mm):
                    uv = jnp.full((L,), u, jnp.int32)
                    x2b = jnp.take_along_axis(
                        x2v, uv, axis=0, mode="promise_in_bounds")
                    y2b = jnp.take_along_axis(
                        y2v, uv, axis=0, mode="promise_in_bounds")
                    n2b = jnp.take_along_axis(
                        n2v, uv, axis=0, mode="promise_in_bounds")
                    return tuple(
                        jnp.minimum(mm[rc], n2b + nx[rc] * x2b + ny[rc] * y2b)
                        for rc in range(CHUNK))

                return lax.fori_loop(0, L, u_body, m, unroll=2)

            m0 = tuple(jnp.full((L,), 3.0e38, jnp.float32)
                       for _ in range(CHUNK))
            m = lax.fori_loop(0, M // L, jj_body, m0)
            for rc in range(CHUNK):
                s_vec = s_vec + m[rc]

        out_v[...] = s_vec
        pltpu.sync_copy(out_v, out_h.at[pl.ds(w * L, L)])

    return k(p1x, p1y, p2x, p2y)


def _tc_body(p1x_r, p1y_r, p2x_r, p2y_r, o_r):
    s = jnp.float32(0.0)
    for rb in range(TC_NRB):
        r0 = SC_X + rb * TC_RB
        x1 = p1x_r[0, 0, pl.ds(r0, TC_RB)]
        y1 = p1y_r[0, 0, pl.ds(r0, TC_RB)]
        x1m2 = (x1 * -2.0)[:, None]
        y1m2 = (y1 * -2.0)[:, None]

        def cb(ci, m2d, x1m2=x1m2, y1m2=y1m2):
            x2 = p2x_r[0, 0, pl.ds(ci * TC_CB, TC_CB)][None, :]
            y2 = p2y_r[0, 0, pl.ds(ci * TC_CB, TC_CB)][None, :]
            n2 = x2 * x2 + y2 * y2
            d = n2 + x1m2 * x2 + y1m2 * y2
            return jnp.minimum(m2d, d)

        m2d = lax.fori_loop(0, M // TC_CB, cb,
                            jnp.full((TC_RB, TC_CB), 3.0e38, jnp.float32))
        m = jnp.min(m2d, axis=1)
        s = s + jnp.sum(m + (x1 * x1 + y1 * y1))
    o_r[0, 0, :] = jnp.full((128,), s, jnp.float32)


def _tc_nn_cost(p1x2, p1y2, p2x2, p2y2):
    a = [v.reshape(B, 1, M) for v in (p1x2, p1y2, p2x2, p2y2)]
    spec = pl.BlockSpec((1, 1, M), lambda b: (b, 0, 0))
    out = pl.pallas_call(
        _tc_body,
        grid=(B,),
        in_specs=[spec, spec, spec, spec],
        out_specs=pl.BlockSpec((1, 1, 128), lambda b: (b, 0, 0)),
        out_shape=jax.ShapeDtypeStruct((B, 1, 128), jnp.float32),
        compiler_params=pltpu.CompilerParams(
            dimension_semantics=("parallel",)),
    )(*a)
    return out[:, 0, 0]


@jax.jit
def kernel(p1, p2):
    p1x2 = p1[:, :, 0]
    p1y2 = p1[:, :, 1]
    p2x2 = p2[:, :, 0]
    p2y2 = p2[:, :, 1]
    tc_part = _tc_nn_cost(p1x2, p1y2, p2x2, p2y2)
    sc_part = _sc_nn_cost(p1x2.reshape(-1), p1y2.reshape(-1),
                          p2x2.reshape(-1), p2y2.reshape(-1))
    return sc_part.reshape(B, W_PER_B * L).sum(axis=1) + tc_part


# speedup vs baseline: 1.7185x; 1.3922x over previous
import functools

import jax
import jax.numpy as jnp
from jax import lax
from jax.experimental import pallas as pl
from jax.experimental.pallas import tpu as pltpu
from jax.experimental.pallas import tpu_sc as plsc

B = 4
M = 2048
NC, NS, L = 2, 16, 16
NW = NC * NS
W_PER_B = NW // B

SC_X = 512
SC_ROWS_PER_W = B * SC_X // NW
CHUNK = 4
PASS_ROWS = CHUNK * L
NPASS = SC_ROWS_PER_W // PASS_ROWS

TC_RB = 512
TC_CB = 1024
TC_NRB = (M - SC_X) // TC_RB


def _sc_nn_cost(p1x, p1y, p2x, p2y):
    mesh = plsc.VectorSubcoreMesh(
        core_axis_name="c", subcore_axis_name="s",
        num_cores=NC, num_subcores=NS)

    @functools.partial(
        pl.kernel,
        out_type=jax.ShapeDtypeStruct((NW * L,), jnp.float32),
        mesh=mesh,
        scratch_types=[
            pltpu.VMEM((SC_ROWS_PER_W,), jnp.float32),
            pltpu.VMEM((SC_ROWS_PER_W,), jnp.float32),
            pltpu.VMEM((M,), jnp.float32),
            pltpu.VMEM((M,), jnp.float32),
            pltpu.VMEM((M,), jnp.float32),
            pltpu.VMEM((L,), jnp.float32),
        ],
    )
    def k(p1x_h, p1y_h, p2x_h, p2y_h, out_h,
          p1x_v, p1y_v, p2x_v, p2y_v, n2_v, out_v):
        c = lax.axis_index("c")
        s = lax.axis_index("s")
        w = c * NS + s
        b = w // W_PER_B
        base = b * M + (w % W_PER_B) * SC_ROWS_PER_W
        pltpu.sync_copy(p1x_h.at[pl.ds(base, SC_ROWS_PER_W)], p1x_v)
        pltpu.sync_copy(p1y_h.at[pl.ds(base, SC_ROWS_PER_W)], p1y_v)
        pltpu.sync_copy(p2x_h.at[pl.ds(b * M, M)], p2x_v)
        pltpu.sync_copy(p2y_h.at[pl.ds(b * M, M)], p2y_v)

        def n2_body(kk, carry):
            x2 = p2x_v[pl.ds(kk * L, L)]
            y2 = p2y_v[pl.ds(kk * L, L)]
            n2_v[pl.ds(kk * L, L)] = x2 * x2 + y2 * y2
            return carry
        lax.fori_loop(0, M // L, n2_body, 0)

        s_vec = jnp.zeros((L,), jnp.float32)
        for p in range(NPASS):
            nx, ny = [], []
            for rc in range(CHUNK):
                off = p * PASS_ROWS + rc * L
                x1 = p1x_v[pl.ds(off, L)]
                y1 = p1y_v[pl.ds(off, L)]
                s_vec = s_vec + (x1 * x1 + y1 * y1)
                nx.append(x1 * -2.0)
                ny.append(y1 * -2.0)

            def jj_body(jj, m, nx=nx, ny=ny):
                base_j = jj * L
                x2v = p2x_v[pl.ds(base_j, L)]
                y2v = p2y_v[pl.ds(base_j, L)]
                n2v = n2_v[pl.ds(base_j, L)]

                def u_body(u, mm):
                    uv = jnp.full((L,), u, jnp.int32)
                    x2b = jnp.take_along_axis(
                        x2v, uv, axis=0, mode="promise_in_bounds")
                    y2b = jnp.take_along_axis(
                        y2v, uv, axis=0, mode="promise_in_bounds")
                    n2b = jnp.take_along_axis(
                        n2v, uv, axis=0, mode="promise_in_bounds")
                    return tuple(
                        jnp.minimum(mm[rc], n2b + nx[rc] * x2b + ny[rc] * y2b)
                        for rc in range(CHUNK))

                return lax.fori_loop(0, L, u_body, m, unroll=2)

            m0 = tuple(jnp.full((L,), 3.0e38, jnp.float32)
                       for _ in range(CHUNK))
            m = lax.fori_loop(0, M // L, jj_body, m0)
            for rc in range(CHUNK):
                s_vec = s_vec + m[rc]

        out_v[...] = s_vec
        pltpu.sync_copy(out_v, out_h.at[pl.ds(w * L, L)])

    return k(p1x, p1y, p2x, p2y)


def _tc_body(p1x_r, p1y_r, p2x_r, p2y_r, o_r):
    s = jnp.float32(0.0)
    for rb in range(TC_NRB):
        r0 = SC_X + rb * TC_RB
        x1 = p1x_r[0, 0, pl.ds(r0, TC_RB)]
        y1 = p1y_r[0, 0, pl.ds(r0, TC_RB)]
        x1m2 = (x1 * -2.0)[:, None]
        y1m2 = (y1 * -2.0)[:, None]

        def cb(ci, m, x1m2=x1m2, y1m2=y1m2):
            x2 = p2x_r[0, 0, pl.ds(ci * TC_CB, TC_CB)][None, :]
            y2 = p2y_r[0, 0, pl.ds(ci * TC_CB, TC_CB)][None, :]
            n2 = x2 * x2 + y2 * y2
            d = n2 + x1m2 * x2 + y1m2 * y2
            return jnp.minimum(m, jnp.min(d, axis=1))

        m = lax.fori_loop(0, M // TC_CB, cb,
                          jnp.full((TC_RB,), 3.0e38, jnp.float32))
        s = s + jnp.sum(m + (x1 * x1 + y1 * y1))
    o_r[0, 0, :] = jnp.full((128,), s, jnp.float32)


def _tc_nn_cost(p1x2, p1y2, p2x2, p2y2):
    a = [v.reshape(B, 1, M) for v in (p1x2, p1y2, p2x2, p2y2)]
    spec = pl.BlockSpec((1, 1, M), lambda b: (b, 0, 0))
    out = pl.pallas_call(
        _tc_body,
        grid=(B,),
        in_specs=[spec, spec, spec, spec],
        out_specs=pl.BlockSpec((1, 1, 128), lambda b: (b, 0, 0)),
        out_shape=jax.ShapeDtypeStruct((B, 1, 128), jnp.float32),
        compiler_params=pltpu.CompilerParams(
            dimension_semantics=("parallel",)),
    )(*a)
    return out[:, 0, 0]


@jax.jit
def kernel(p1, p2):
    p1x2 = p1[:, :, 0]
    p1y2 = p1[:, :, 1]
    p2x2 = p2[:, :, 0]
    p2y2 = p2[:, :, 1]
    tc_part = _tc_nn_cost(p1x2, p1y2, p2x2, p2y2)
    sc_part = _sc_nn_cost(p1x2.reshape(-1), p1y2.reshape(-1),
                          p2x2.reshape(-1), p2y2.reshape(-1))
    return sc_part.reshape(B, W_PER_B * L).sum(axis=1) + tc_part


# speedup vs baseline: 1.7768x; 1.0339x over previous
import functools

import jax
import jax.numpy as jnp
from jax import lax
from jax.experimental import pallas as pl
from jax.experimental.pallas import tpu as pltpu
from jax.experimental.pallas import tpu_sc as plsc

B = 4
M = 2048
NC, NS, L = 2, 16, 16
NW = NC * NS
W_PER_B = NW // B

SC_X = 512
SC_ROWS_PER_W = B * SC_X // NW
CHUNK = 4
PASS_ROWS = CHUNK * L
NPASS = SC_ROWS_PER_W // PASS_ROWS

TC_RB = 512
TC_CB = 2048
TC_NRB = (M - SC_X) // TC_RB


def _sc_nn_cost(p1x, p1y, p2x, p2y):
    mesh = plsc.VectorSubcoreMesh(
        core_axis_name="c", subcore_axis_name="s",
        num_cores=NC, num_subcores=NS)

    @functools.partial(
        pl.kernel,
        out_type=jax.ShapeDtypeStruct((NW * L,), jnp.float32),
        mesh=mesh,
        scratch_types=[
            pltpu.VMEM((SC_ROWS_PER_W,), jnp.float32),
            pltpu.VMEM((SC_ROWS_PER_W,), jnp.float32),
            pltpu.VMEM((M,), jnp.float32),
            pltpu.VMEM((M,), jnp.float32),
            pltpu.VMEM((M,), jnp.float32),
            pltpu.VMEM((L,), jnp.float32),
        ],
    )
    def k(p1x_h, p1y_h, p2x_h, p2y_h, out_h,
          p1x_v, p1y_v, p2x_v, p2y_v, n2_v, out_v):
        c = lax.axis_index("c")
        s = lax.axis_index("s")
        w = c * NS + s
        b = w // W_PER_B
        base = b * M + (w % W_PER_B) * SC_ROWS_PER_W
        pltpu.sync_copy(p1x_h.at[pl.ds(base, SC_ROWS_PER_W)], p1x_v)
        pltpu.sync_copy(p1y_h.at[pl.ds(base, SC_ROWS_PER_W)], p1y_v)
        pltpu.sync_copy(p2x_h.at[pl.ds(b * M, M)], p2x_v)
        pltpu.sync_copy(p2y_h.at[pl.ds(b * M, M)], p2y_v)

        def n2_body(kk, carry):
            x2 = p2x_v[pl.ds(kk * L, L)]
            y2 = p2y_v[pl.ds(kk * L, L)]
            n2_v[pl.ds(kk * L, L)] = x2 * x2 + y2 * y2
            return carry
        lax.fori_loop(0, M // L, n2_body, 0)

        s_vec = jnp.zeros((L,), jnp.float32)
        for p in range(NPASS):
            nx, ny = [], []
            for rc in range(CHUNK):
                off = p * PASS_ROWS + rc * L
                x1 = p1x_v[pl.ds(off, L)]
                y1 = p1y_v[pl.ds(off, L)]
                s_vec = s_vec + (x1 * x1 + y1 * y1)
                nx.append(x1 * -2.0)
                ny.append(y1 * -2.0)

            def jj_body(jj, m, nx=nx, ny=ny):
                base_j = jj * L
                x2v = p2x_v[pl.ds(base_j, L)]
                y2v = p2y_v[pl.ds(base_j, L)]
                n2v = n2_v[pl.ds(base_j, L)]

                def u_body(u, mm):
                    uv = jnp.full((L,), u, jnp.int32)
                    x2b = jnp.take_along_axis(
                        x2v, uv, axis=0, mode="promise_in_bounds")
                    y2b = jnp.take_along_axis(
                        y2v, uv, axis=0, mode="promise_in_bounds")
                    n2b = jnp.take_along_axis(
                        n2v, uv, axis=0, mode="promise_in_bounds")
                    return tuple(
                        jnp.minimum(mm[rc], n2b + nx[rc] * x2b + ny[rc] * y2b)
                        for rc in range(CHUNK))

                return lax.fori_loop(0, L, u_body, m, unroll=2)

            m0 = tuple(jnp.full((L,), 3.0e38, jnp.float32)
                       for _ in range(CHUNK))
            m = lax.fori_loop(0, M // L, jj_body, m0)
            for rc in range(CHUNK):
                s_vec = s_vec + m[rc]

        out_v[...] = s_vec
        pltpu.sync_copy(out_v, out_h.at[pl.ds(w * L, L)])

    return k(p1x, p1y, p2x, p2y)


def _tc_body(p1x_r, p1y_r, p2x_r, p2y_r, o_r):
    s = jnp.float32(0.0)
    for rb in range(TC_NRB):
        r0 = SC_X + rb * TC_RB
        x1 = p1x_r[0, 0, pl.ds(r0, TC_RB)]
        y1 = p1y_r[0, 0, pl.ds(r0, TC_RB)]
        x1m2 = (x1 * -2.0)[:, None]
        y1m2 = (y1 * -2.0)[:, None]

        def cb(ci, m, x1m2=x1m2, y1m2=y1m2):
            x2 = p2x_r[0, 0, pl.ds(ci * TC_CB, TC_CB)][None, :]
            y2 = p2y_r[0, 0, pl.ds(ci * TC_CB, TC_CB)][None, :]
            n2 = x2 * x2 + y2 * y2
            d = n2 + x1m2 * x2 + y1m2 * y2
            return jnp.minimum(m, jnp.min(d, axis=1))

        m = lax.fori_loop(0, M // TC_CB, cb,
                          jnp.full((TC_RB,), 3.0e38, jnp.float32))
        s = s + jnp.sum(m + (x1 * x1 + y1 * y1))
    o_r[0, 0, :] = jnp.full((128,), s, jnp.float32)


def _tc_nn_cost(p1x2, p1y2, p2x2, p2y2):
    a = [v.reshape(B, 1, M) for v in (p1x2, p1y2, p2x2, p2y2)]
    spec = pl.BlockSpec((1, 1, M), lambda b: (b, 0, 0))
    out = pl.pallas_call(
        _tc_body,
        grid=(B,),
        in_specs=[spec, spec, spec, spec],
        out_specs=pl.BlockSpec((1, 1, 128), lambda b: (b, 0, 0)),
        out_shape=jax.ShapeDtypeStruct((B, 1, 128), jnp.float32),
        compiler_params=pltpu.CompilerParams(
            dimension_semantics=("parallel",)),
    )(*a)
    return out[:, 0, 0]


@jax.jit
def kernel(p1, p2):
    p1x2 = p1[:, :, 0]
    p1y2 = p1[:, :, 1]
    p2x2 = p2[:, :, 0]
    p2y2 = p2[:, :, 1]
    tc_part = _tc_nn_cost(p1x2, p1y2, p2x2, p2y2)
    sc_part = _sc_nn_cost(p1x2.reshape(-1), p1y2.reshape(-1),
                          p2x2.reshape(-1), p2y2.reshape(-1))
    return sc_part.reshape(B, W_PER_B * L).sum(axis=1) + tc_part


# speedup vs baseline: 1.7799x; 1.0017x over previous
import functools

import jax
import jax.numpy as jnp
from jax import lax
from jax.experimental import pallas as pl
from jax.experimental.pallas import tpu as pltpu
from jax.experimental.pallas import tpu_sc as plsc

B = 4
M = 2048
NC, NS, L = 2, 16, 16
NW = NC * NS
W_PER_B = NW // B

SC_X = 512
SC_ROWS_PER_W = B * SC_X // NW
CHUNK = 4
PASS_ROWS = CHUNK * L
NPASS = SC_ROWS_PER_W // PASS_ROWS

TC_RB = 768
TC_CB = 2048
TC_NRB = (M - SC_X) // TC_RB


def _sc_nn_cost(p1x, p1y, p2x, p2y):
    mesh = plsc.VectorSubcoreMesh(
        core_axis_name="c", subcore_axis_name="s",
        num_cores=NC, num_subcores=NS)

    @functools.partial(
        pl.kernel,
        out_type=jax.ShapeDtypeStruct((NW * L,), jnp.float32),
        mesh=mesh,
        scratch_types=[
            pltpu.VMEM((SC_ROWS_PER_W,), jnp.float32),
            pltpu.VMEM((SC_ROWS_PER_W,), jnp.float32),
            pltpu.VMEM((M,), jnp.float32),
            pltpu.VMEM((M,), jnp.float32),
            pltpu.VMEM((M,), jnp.float32),
            pltpu.VMEM((L,), jnp.float32),
        ],
    )
    def k(p1x_h, p1y_h, p2x_h, p2y_h, out_h,
          p1x_v, p1y_v, p2x_v, p2y_v, n2_v, out_v):
        c = lax.axis_index("c")
        s = lax.axis_index("s")
        w = c * NS + s
        b = w // W_PER_B
        base = b * M + (w % W_PER_B) * SC_ROWS_PER_W
        pltpu.sync_copy(p1x_h.at[pl.ds(base, SC_ROWS_PER_W)], p1x_v)
        pltpu.sync_copy(p1y_h.at[pl.ds(base, SC_ROWS_PER_W)], p1y_v)
        pltpu.sync_copy(p2x_h.at[pl.ds(b * M, M)], p2x_v)
        pltpu.sync_copy(p2y_h.at[pl.ds(b * M, M)], p2y_v)

        def n2_body(kk, carry):
            x2 = p2x_v[pl.ds(kk * L, L)]
            y2 = p2y_v[pl.ds(kk * L, L)]
            n2_v[pl.ds(kk * L, L)] = x2 * x2 + y2 * y2
            return carry
        lax.fori_loop(0, M // L, n2_body, 0)

        s_vec = jnp.zeros((L,), jnp.float32)
        for p in range(NPASS):
            nx, ny = [], []
            for rc in range(CHUNK):
                off = p * PASS_ROWS + rc * L
                x1 = p1x_v[pl.ds(off, L)]
                y1 = p1y_v[pl.ds(off, L)]
                s_vec = s_vec + (x1 * x1 + y1 * y1)
                nx.append(x1 * -2.0)
                ny.append(y1 * -2.0)

            def jj_body(jj, m, nx=nx, ny=ny):
                base_j = jj * L
                x2v = p2x_v[pl.ds(base_j, L)]
                y2v = p2y_v[pl.ds(base_j, L)]
                n2v = n2_v[pl.ds(base_j, L)]

                def u_body(u, mm):
                    uv = jnp.full((L,), u, jnp.int32)
                    x2b = jnp.take_along_axis(
                        x2v, uv, axis=0, mode="promise_in_bounds")
                    y2b = jnp.take_along_axis(
                        y2v, uv, axis=0, mode="promise_in_bounds")
                    n2b = jnp.take_along_axis(
                        n2v, uv, axis=0, mode="promise_in_bounds")
                    return tuple(
                        jnp.minimum(mm[rc], n2b + nx[rc] * x2b + ny[rc] * y2b)
                        for rc in range(CHUNK))

                return lax.fori_loop(0, L, u_body, m, unroll=2)

            m0 = tuple(jnp.full((L,), 3.0e38, jnp.float32)
                       for _ in range(CHUNK))
            m = lax.fori_loop(0, M // L, jj_body, m0)
            for rc in range(CHUNK):
                s_vec = s_vec + m[rc]

        out_v[...] = s_vec
        pltpu.sync_copy(out_v, out_h.at[pl.ds(w * L, L)])

    return k(p1x, p1y, p2x, p2y)


def _tc_body(p1x_r, p1y_r, p2x_r, p2y_r, o_r):
    s = jnp.float32(0.0)
    for rb in range(TC_NRB):
        r0 = SC_X + rb * TC_RB
        x1 = p1x_r[0, 0, pl.ds(r0, TC_RB)]
        y1 = p1y_r[0, 0, pl.ds(r0, TC_RB)]
        x1m2 = (x1 * -2.0)[:, None]
        y1m2 = (y1 * -2.0)[:, None]

        def cb(ci, m, x1m2=x1m2, y1m2=y1m2):
            x2 = p2x_r[0, 0, pl.ds(ci * TC_CB, TC_CB)][None, :]
            y2 = p2y_r[0, 0, pl.ds(ci * TC_CB, TC_CB)][None, :]
            n2 = x2 * x2 + y2 * y2
            d = n2 + x1m2 * x2 + y1m2 * y2
            return jnp.minimum(m, jnp.min(d, axis=1))

        m = lax.fori_loop(0, M // TC_CB, cb,
                          jnp.full((TC_RB,), 3.0e38, jnp.float32))
        s = s + jnp.sum(m + (x1 * x1 + y1 * y1))
    o_r[0, 0, :] = jnp.full((128,), s, jnp.float32)


def _tc_nn_cost(p1x2, p1y2, p2x2, p2y2):
    a = [v.reshape(B, 1, M) for v in (p1x2, p1y2, p2x2, p2y2)]
    spec = pl.BlockSpec((1, 1, M), lambda b: (b, 0, 0))
    out = pl.pallas_call(
        _tc_body,
        grid=(B,),
        in_specs=[spec, spec, spec, spec],
        out_specs=pl.BlockSpec((1, 1, 128), lambda b: (b, 0, 0)),
        out_shape=jax.ShapeDtypeStruct((B, 1, 128), jnp.float32),
        compiler_params=pltpu.CompilerParams(
            dimension_semantics=("parallel",)),
    )(*a)
    return out[:, 0, 0]


@jax.jit
def kernel(p1, p2):
    p1x2 = p1[:, :, 0]
    p1y2 = p1[:, :, 1]
    p2x2 = p2[:, :, 0]
    p2y2 = p2[:, :, 1]
    tc_part = _tc_nn_cost(p1x2, p1y2, p2x2, p2y2)
    sc_part = _sc_nn_cost(p1x2.reshape(-1), p1y2.reshape(-1),
                          p2x2.reshape(-1), p2y2.reshape(-1))
    return sc_part.reshape(B, W_PER_B * L).sum(axis=1) + tc_part


# speedup vs baseline: 1.8148x; 1.0196x over previous
import functools

import jax
import jax.numpy as jnp
from jax import lax
from jax.experimental import pallas as pl
from jax.experimental.pallas import tpu as pltpu
from jax.experimental.pallas import tpu_sc as plsc

B = 4
M = 2048
NC, NS, L = 2, 16, 16
NW = NC * NS
W_PER_B = NW // B

SC_X = 512
SC_ROWS_PER_W = B * SC_X // NW
CHUNK = 4
PASS_ROWS = CHUNK * L
NPASS = SC_ROWS_PER_W // PASS_ROWS

TC_RB = 768
TC_CB = 2048
TC_NRB = (M - SC_X) // TC_RB


def _sc_nn_cost(p1x, p1y, p2x, p2y):
    mesh = plsc.VectorSubcoreMesh(
        core_axis_name="c", subcore_axis_name="s",
        num_cores=NC, num_subcores=NS)

    @functools.partial(
        pl.kernel,
        out_type=jax.ShapeDtypeStruct((NW * L,), jnp.float32),
        mesh=mesh,
        scratch_types=[
            pltpu.VMEM((SC_ROWS_PER_W,), jnp.float32),
            pltpu.VMEM((SC_ROWS_PER_W,), jnp.float32),
            pltpu.VMEM((M,), jnp.float32),
            pltpu.VMEM((M,), jnp.float32),
            pltpu.VMEM((M,), jnp.float32),
            pltpu.VMEM((L,), jnp.float32),
        ],
    )
    def k(p1x_h, p1y_h, p2x_h, p2y_h, out_h,
          p1x_v, p1y_v, p2x_v, p2y_v, n2_v, out_v):
        c = lax.axis_index("c")
        s = lax.axis_index("s")
        w = c * NS + s
        b = w // W_PER_B
        base = b * M + (w % W_PER_B) * SC_ROWS_PER_W
        pltpu.sync_copy(p1x_h.at[pl.ds(base, SC_ROWS_PER_W)], p1x_v)
        pltpu.sync_copy(p1y_h.at[pl.ds(base, SC_ROWS_PER_W)], p1y_v)
        pltpu.sync_copy(p2x_h.at[pl.ds(b * M, M)], p2x_v)
        pltpu.sync_copy(p2y_h.at[pl.ds(b * M, M)], p2y_v)

        def n2_body(kk, carry):
            x2 = p2x_v[pl.ds(kk * L, L)]
            y2 = p2y_v[pl.ds(kk * L, L)]
            n2_v[pl.ds(kk * L, L)] = x2 * x2 + y2 * y2
            return carry
        lax.fori_loop(0, M // L, n2_body, 0)

        s_vec = jnp.zeros((L,), jnp.float32)
        for p in range(NPASS):
            nx, ny = [], []
            for rc in range(CHUNK):
                off = p * PASS_ROWS + rc * L
                x1 = p1x_v[pl.ds(off, L)]
                y1 = p1y_v[pl.ds(off, L)]
                s_vec = s_vec + (x1 * x1 + y1 * y1)
                nx.append(x1 * -2.0)
                ny.append(y1 * -2.0)

            def jj_body(jj, m, nx=nx, ny=ny):
                base_j = jj * L
                x2v = p2x_v[pl.ds(base_j, L)]
                y2v = p2y_v[pl.ds(base_j, L)]
                n2v = n2_v[pl.ds(base_j, L)]

                def u_body(u, mm):
                    uv = jnp.full((L,), u, jnp.int32)
                    x2b = jnp.take_along_axis(
                        x2v, uv, axis=0, mode="promise_in_bounds")
                    y2b = jnp.take_along_axis(
                        y2v, uv, axis=0, mode="promise_in_bounds")
                    n2b = jnp.take_along_axis(
                        n2v, uv, axis=0, mode="promise_in_bounds")
                    return tuple(
                        jnp.minimum(mm[rc], n2b + nx[rc] * x2b + ny[rc] * y2b)
                        for rc in range(CHUNK))

                return lax.fori_loop(0, L, u_body, m, unroll=4)

            m0 = tuple(jnp.full((L,), 3.0e38, jnp.float32)
                       for _ in range(CHUNK))
            m = lax.fori_loop(0, M // L, jj_body, m0)
            for rc in range(CHUNK):
                s_vec = s_vec + m[rc]

        out_v[...] = s_vec
        pltpu.sync_copy(out_v, out_h.at[pl.ds(w * L, L)])

    return k(p1x, p1y, p2x, p2y)


def _tc_body(p1x_r, p1y_r, p2x_r, p2y_r, o_r):
    s = jnp.float32(0.0)
    for rb in range(TC_NRB):
        r0 = SC_X + rb * TC_RB
        x1 = p1x_r[0, 0, pl.ds(r0, TC_RB)]
        y1 = p1y_r[0, 0, pl.ds(r0, TC_RB)]
        x1m2 = (x1 * -2.0)[:, None]
        y1m2 = (y1 * -2.0)[:, None]

        def cb(ci, m, x1m2=x1m2, y1m2=y1m2):
            x2 = p2x_r[0, 0, pl.ds(ci * TC_CB, TC_CB)][None, :]
            y2 = p2y_r[0, 0, pl.ds(ci * TC_CB, TC_CB)][None, :]
            n2 = x2 * x2 + y2 * y2
            d = n2 + x1m2 * x2 + y1m2 * y2
            return jnp.minimum(m, jnp.min(d, axis=1))

        m = lax.fori_loop(0, M // TC_CB, cb,
                          jnp.full((TC_RB,), 3.0e38, jnp.float32))
        s = s + jnp.sum(m + (x1 * x1 + y1 * y1))
    o_r[0, 0, :] = jnp.full((128,), s, jnp.float32)


def _tc_nn_cost(p1x2, p1y2, p2x2, p2y2):
    a = [v.reshape(B, 1, M) for v in (p1x2, p1y2, p2x2, p2y2)]
    spec = pl.BlockSpec((1, 1, M), lambda b: (b, 0, 0))
    out = pl.pallas_call(
        _tc_body,
        grid=(B,),
        in_specs=[spec, spec, spec, spec],
        out_specs=pl.BlockSpec((1, 1, 128), lambda b: (b, 0, 0)),
        out_shape=jax.ShapeDtypeStruct((B, 1, 128), jnp.float32),
        compiler_params=pltpu.CompilerParams(
            dimension_semantics=("parallel",)),
    )(*a)
    return out[:, 0, 0]


@jax.jit
def kernel(p1, p2):
    p1x2 = p1[:, :, 0]
    p1y2 = p1[:, :, 1]
    p2x2 = p2[:, :, 0]
    p2y2 = p2[:, :, 1]
    tc_part = _tc_nn_cost(p1x2, p1y2, p2x2, p2y2)
    sc_part = _sc_nn_cost(p1x2.reshape(-1), p1y2.reshape(-1),
                          p2x2.reshape(-1), p2y2.reshape(-1))
    return sc_part.reshape(B, W_PER_B * L).sum(axis=1) + tc_part
